# baseline (device time: 230831 ns/iter reference)
import jax
import jax.numpy as jnp
from jax import lax
from jax.experimental import pallas as pl
from jax.experimental.pallas import tpu as pltpu

N_DEV = 8


def kernel(x, w_mat, scale_x, scale_w):
    m_per, k = x.shape
    _, n_per = w_mat.shape

    x8 = x.astype(jnp.float8_e4m3fn)
    w8 = w_mat.astype(jnp.float8_e5m2)

    dot_dims = (((1,), (0,)), ((), ()))

    def body(x_ref, w_ref, sx_ref, sw_ref, out_ref, comm_ref,
             send_sems, recv_sems):
        my = lax.axis_index("i")
        left = lax.rem(my + (N_DEV - 1), N_DEV)
        right = lax.rem(my + 1, N_DEV)

        barrier_sem = pltpu.get_barrier_semaphore()
        for nbr in (left, right):
            pl.semaphore_signal(
                barrier_sem, inc=1,
                device_id=(nbr,), device_id_type=pl.DeviceIdType.MESH,
            )
        pl.semaphore_wait(barrier_sem, 2)

        scale = sx_ref[0] * sw_ref[0]

        def gemm_store(chunk, origin):
            acc = lax.dot_general(
                chunk, w_ref[...], dot_dims,
                preferred_element_type=jnp.float32,
            )
            out_ref[pl.ds(origin * m_per, m_per), :] = jnp.maximum(
                acc * scale, 0.0
            )

        comm_ref[0] = x_ref[...]
        gemm_store(x_ref[...], my)

        for h in range(N_DEV - 1):
            rdma = pltpu.make_async_remote_copy(
                src_ref=comm_ref.at[h],
                dst_ref=comm_ref.at[h + 1],
                send_sem=send_sems.at[h],
                recv_sem=recv_sems.at[h],
                device_id=(right,),
                device_id_type=pl.DeviceIdType.MESH,
            )
            rdma.start()
            rdma.wait()
            origin = lax.rem(my + (N_DEV - 1 - h), N_DEV)
            gemm_store(comm_ref[h + 1], origin)

    return pl.pallas_call(
        body,
        out_shape=jax.ShapeDtypeStruct((N_DEV * m_per, n_per), jnp.float32),
        in_specs=[
            pl.BlockSpec(memory_space=pltpu.VMEM),
            pl.BlockSpec(memory_space=pltpu.VMEM),
            pl.BlockSpec(memory_space=pltpu.SMEM),
            pl.BlockSpec(memory_space=pltpu.SMEM),
        ],
        out_specs=pl.BlockSpec(memory_space=pltpu.VMEM),
        scratch_shapes=[
            pltpu.VMEM((N_DEV, m_per, k), jnp.float8_e4m3fn),
            pltpu.SemaphoreType.DMA((N_DEV - 1,)),
            pltpu.SemaphoreType.DMA((N_DEV - 1,)),
        ],
        compiler_params=pltpu.CompilerParams(
            collective_id=0,
            vmem_limit_bytes=100 * 1024 * 1024,
        ),
    )(x8, w8, scale_x, scale_w)


# device time: 118896 ns/iter; 1.9415x vs baseline; 1.9415x over previous
import jax
import jax.numpy as jnp
from jax import lax
from jax.experimental import pallas as pl
from jax.experimental.pallas import tpu as pltpu

N_DEV = 8
M_PER = 512
K = 4096

S_OFF = (0, 192, 352)
S_ROWS = (192, 160, 160)
MASKS = ((1, 3, 4), (3, 4, 1), (4, 1, 3))
OFFS = tuple(
    ([0], [0, m[0]], [0, m[0], m[1], m[0] ^ m[1]]) for m in MASKS
)


def kernel(x, w_mat, scale_x, scale_w):
    m_per, k = x.shape
    _, n_per = w_mat.shape
    assert (m_per, k) == (M_PER, K)

    w8 = w_mat.astype(jnp.float8_e5m2)

    dot_dims = (((1,), (0,)), ((), ()))

    def body(x_ref, w_ref, sx_ref, sw_ref, out_ref, comm_ref,
             send_sems, recv_sems):
        my = lax.axis_index("i")

        barrier_sem = pltpu.get_barrier_semaphore()
        for mask in (1, 3, 4):
            pl.semaphore_signal(
                barrier_sem, inc=1,
                device_id=(my ^ mask,), device_id_type=pl.DeviceIdType.MESH,
            )
        pl.semaphore_wait(barrier_sem, 3)

        comm_ref[pl.ds(my * M_PER, M_PER), :] = x_ref[...].astype(
            jnp.float8_e4m3fn
        )

        for r in range(3):
            rdmas = []
            for s in range(3):
                partner = my ^ MASKS[s][r]
                for off in OFFS[s][r]:
                    q = my ^ off
                    row0 = q * M_PER + S_OFF[s]
                    rdma = pltpu.make_async_remote_copy(
                        src_ref=comm_ref.at[pl.ds(row0, S_ROWS[s]), :],
                        dst_ref=comm_ref.at[pl.ds(row0, S_ROWS[s]), :],
                        send_sem=send_sems.at[s, r],
                        recv_sem=recv_sems.at[s, r],
                        device_id=(partner,),
                        device_id_type=pl.DeviceIdType.MESH,
                    )
                    rdma.start()
                    rdmas.append(rdma)
            for rdma in rdmas:
                rdma.wait()

        scale = sx_ref[0] * sw_ref[0]
        for c in range(N_DEV):
            acc = lax.dot_general(
                comm_ref[pl.ds(c * M_PER, M_PER), :], w_ref[...], dot_dims,
                preferred_element_type=jnp.float32,
            )
            out_ref[pl.ds(c * M_PER, M_PER), :] = jnp.maximum(
                acc * scale, 0.0
            )

    return pl.pallas_call(
        body,
        out_shape=jax.ShapeDtypeStruct((N_DEV * m_per, n_per), jnp.float32),
        in_specs=[
            pl.BlockSpec(memory_space=pltpu.VMEM),
            pl.BlockSpec(memory_space=pltpu.VMEM),
            pl.BlockSpec(memory_space=pltpu.SMEM),
            pl.BlockSpec(memory_space=pltpu.SMEM),
        ],
        out_specs=pl.BlockSpec(memory_space=pltpu.VMEM),
        scratch_shapes=[
            pltpu.VMEM((N_DEV * M_PER, K), jnp.float8_e4m3fn),
            pltpu.SemaphoreType.DMA((3, 3)),
            pltpu.SemaphoreType.DMA((3, 3)),
        ],
        compiler_params=pltpu.CompilerParams(
            collective_id=0,
            vmem_limit_bytes=100 * 1024 * 1024,
        ),
    )(x, w8, scale_x, scale_w)


# device time: 101923 ns/iter; 2.2648x vs baseline; 1.1665x over previous
import jax
import jax.numpy as jnp
from jax import lax
from jax.experimental import pallas as pl
from jax.experimental.pallas import tpu as pltpu

N_DEV = 8
M_PER = 512
K = 4096

S_OFF = (0, 192, 352)
S_ROWS = (192, 160, 160)
MASKS = ((1, 3, 4), (3, 4, 1), (4, 1, 3))
OFFS = tuple(
    ([0], [0, m[0]], [0, m[0], m[1], m[0] ^ m[1]]) for m in MASKS
)


def kernel(x, w_mat, scale_x, scale_w):
    m_per, k = x.shape
    _, n_per = w_mat.shape
    assert (m_per, k) == (M_PER, K)

    w8 = w_mat.astype(jnp.float8_e5m2)

    dot_dims = (((1,), (0,)), ((), ()))

    def body(x_ref, w_ref, sx_ref, sw_ref, out_ref, comm_ref,
             send_sems, recv_sems, recv3_sems):
        my = lax.axis_index("i")

        barrier_sem = pltpu.get_barrier_semaphore()
        for mask in (1, 3, 4):
            pl.semaphore_signal(
                barrier_sem, inc=1,
                device_id=(my ^ mask,), device_id_type=pl.DeviceIdType.MESH,
            )
        pl.semaphore_wait(barrier_sem, 3)

        scale = sx_ref[0] * sw_ref[0]

        def gemm_rows(row0, rows):
            acc = lax.dot_general(
                comm_ref[pl.ds(row0, rows), :], w_ref[...], dot_dims,
                preferred_element_type=jnp.float32,
            )
            out_ref[pl.ds(row0, rows), :] = jnp.maximum(acc * scale, 0.0)

        def stripe_gemm(s, q):
            gemm_rows(q * M_PER + S_OFF[s], S_ROWS[s])

        def make_rdma(s, r, off, j=None):
            q = my ^ off
            row0 = q * M_PER + S_OFF[s]
            recv_sem = (
                recv3_sems.at[s, j] if r == 2 else recv_sems.at[s, r]
            )
            return pltpu.make_async_remote_copy(
                src_ref=comm_ref.at[pl.ds(row0, S_ROWS[s]), :],
                dst_ref=comm_ref.at[pl.ds(row0, S_ROWS[s]), :],
                send_sem=send_sems.at[s, r],
                recv_sem=recv_sem,
                device_id=(my ^ MASKS[s][r],),
                device_id_type=pl.DeviceIdType.MESH,
            )

        comm_ref[pl.ds(my * M_PER, M_PER), :] = x_ref[...].astype(
            jnp.float8_e4m3fn
        )

        r0 = []
        for s in range(3):
            rdma = make_rdma(s, 0, 0)
            rdma.start()
            r0.append(rdma)
        gemm_rows(my * M_PER, M_PER)
        for rdma in r0:
            rdma.wait()

        r1 = []
        for s in range(3):
            for off in OFFS[s][1]:
                rdma = make_rdma(s, 1, off)
                rdma.start()
                r1.append(rdma)
        for s in range(3):
            stripe_gemm(s, my ^ MASKS[s][0])
        for rdma in r1:
            rdma.wait()

        r2 = [[None] * 4 for _ in range(3)]
        for s in range(3):
            for j, off in enumerate(OFFS[s][2]):
                rdma = make_rdma(s, 2, off, j=j)
                rdma.start()
                r2[s][j] = rdma
        for s in range(3):
            m0, m1 = MASKS[s][0], MASKS[s][1]
            stripe_gemm(s, my ^ m1)
            stripe_gemm(s, my ^ m0 ^ m1)
        for j in range(4):
            for s in (1, 2, 0):
                r2[s][j].wait()
                stripe_gemm(s, my ^ MASKS[s][2] ^ OFFS[s][2][j])

    return pl.pallas_call(
        body,
        out_shape=jax.ShapeDtypeStruct((N_DEV * m_per, n_per), jnp.float32),
        in_specs=[
            pl.BlockSpec(memory_space=pltpu.VMEM),
            pl.BlockSpec(memory_space=pltpu.VMEM),
            pl.BlockSpec(memory_space=pltpu.SMEM),
            pl.BlockSpec(memory_space=pltpu.SMEM),
        ],
        out_specs=pl.BlockSpec(memory_space=pltpu.VMEM),
        scratch_shapes=[
            pltpu.VMEM((N_DEV * M_PER, K), jnp.float8_e4m3fn),
            pltpu.SemaphoreType.DMA((3, 3)),
            pltpu.SemaphoreType.DMA((3, 2)),
            pltpu.SemaphoreType.DMA((3, 4)),
        ],
        compiler_params=pltpu.CompilerParams(
            collective_id=0,
            vmem_limit_bytes=100 * 1024 * 1024,
        ),
    )(x, w8, scale_x, scale_w)


# device time: 99646 ns/iter; 2.3165x vs baseline; 1.0229x over previous
import jax
import jax.numpy as jnp
from jax import lax
from jax.experimental import pallas as pl
from jax.experimental.pallas import tpu as pltpu

N_DEV = 8
M_PER = 512
K = 4096

S_OFF = (0, 192, 352)
S_ROWS = (192, 160, 160)
MASKS = ((1, 3, 4), (3, 4, 1), (4, 1, 3))
OFFS = tuple(
    ([0], [0, m[0]], [0, m[0], m[1], m[0] ^ m[1]]) for m in MASKS
)


def kernel(x, w_mat, scale_x, scale_w):
    m_per, k = x.shape
    _, n_per = w_mat.shape
    assert (m_per, k) == (M_PER, K)

    x8 = x.astype(jnp.float8_e4m3fn)

    dot_dims = (((1,), (0,)), ((), ()))

    def body(x_ref, w_ref, sx_ref, sw_ref, out_ref, comm_ref, w8_ref,
             send_sems, recv_sems, recv3_sems):
        my = lax.axis_index("i")

        barrier_sem = pltpu.get_barrier_semaphore()
        for mask in (1, 3, 4):
            pl.semaphore_signal(
                barrier_sem, inc=1,
                device_id=(my ^ mask,), device_id_type=pl.DeviceIdType.MESH,
            )
        pl.semaphore_wait(barrier_sem, 3)

        scale = sx_ref[0] * sw_ref[0]

        def stripe_ref(off, s):
            if off == 0:
                return x_ref.at[pl.ds(S_OFF[s], S_ROWS[s]), :]
            return comm_ref.at[off - 1, pl.ds(S_OFF[s], S_ROWS[s]), :]

        def make_rdma(s, r, off, j=None):
            m = MASKS[s][r]
            recv_sem = (
                recv3_sems.at[s, j] if r == 2 else recv_sems.at[s, r]
            )
            return pltpu.make_async_remote_copy(
                src_ref=stripe_ref(off, s),
                dst_ref=stripe_ref(off ^ m, s),
                send_sem=send_sems.at[s, r],
                recv_sem=recv_sem,
                device_id=(my ^ m,),
                device_id_type=pl.DeviceIdType.MESH,
            )

        def stripe_gemm(s, off):
            acc = lax.dot_general(
                stripe_ref(off, s)[...], w8_ref[...], dot_dims,
                preferred_element_type=jnp.float32,
            )
            row0 = (my ^ off) * M_PER + S_OFF[s]
            out_ref[pl.ds(row0, S_ROWS[s]), :] = jnp.maximum(
                acc * scale, 0.0
            )

        r0 = []
        for s in range(3):
            rdma = make_rdma(s, 0, 0)
            rdma.start()
            r0.append(rdma)

        w8_ref[...] = w_ref[...].astype(jnp.float8_e5m2)
        acc = lax.dot_general(
            x_ref[...], w8_ref[...], dot_dims,
            preferred_element_type=jnp.float32,
        )
        out_ref[pl.ds(my * M_PER, M_PER), :] = jnp.maximum(acc * scale, 0.0)

        for rdma in r0:
            rdma.wait()

        r1 = []
        for s in range(3):
            for off in OFFS[s][1]:
                rdma = make_rdma(s, 1, off)
                rdma.start()
                r1.append(rdma)
        for s in range(3):
            stripe_gemm(s, MASKS[s][0])
        for rdma in r1:
            rdma.wait()

        r2 = [[None] * 4 for _ in range(3)]
        for s in range(3):
            for j, off in enumerate(OFFS[s][2]):
                rdma = make_rdma(s, 2, off, j=j)
                rdma.start()
                r2[s][j] = rdma
        for s in range(3):
            m0, m1 = MASKS[s][0], MASKS[s][1]
            stripe_gemm(s, m1)
            stripe_gemm(s, m0 ^ m1)
        for j in range(4):
            for s in (1, 2, 0):
                r2[s][j].wait()
                stripe_gemm(s, MASKS[s][2] ^ OFFS[s][2][j])

    return pl.pallas_call(
        body,
        out_shape=jax.ShapeDtypeStruct((N_DEV * m_per, n_per), jnp.float32),
        in_specs=[
            pl.BlockSpec(memory_space=pltpu.VMEM),
            pl.BlockSpec(memory_space=pltpu.VMEM),
            pl.BlockSpec(memory_space=pltpu.SMEM),
            pl.BlockSpec(memory_space=pltpu.SMEM),
        ],
        out_specs=pl.BlockSpec(memory_space=pltpu.VMEM),
        scratch_shapes=[
            pltpu.VMEM((N_DEV - 1, M_PER, K), jnp.float8_e4m3fn),
            pltpu.VMEM((K, n_per), jnp.float8_e5m2),
            pltpu.SemaphoreType.DMA((3, 3)),
            pltpu.SemaphoreType.DMA((3, 2)),
            pltpu.SemaphoreType.DMA((3, 4)),
        ],
        compiler_params=pltpu.CompilerParams(
            collective_id=0,
            vmem_limit_bytes=100 * 1024 * 1024,
        ),
    )(x8, w_mat, scale_x, scale_w)


# device time: 91162 ns/iter; 2.5321x vs baseline; 1.0931x over previous
import jax
import jax.numpy as jnp
from jax import lax
from jax.experimental import pallas as pl
from jax.experimental.pallas import tpu as pltpu

N_DEV = 8
M_PER = 512
K = 4096

S_OFF = (0, 192, 352)
S_ROWS = (192, 160, 160)
MASKS = ((1, 3, 4), (3, 4, 1), (4, 1, 3))
OFFS = tuple(
    ([0], [0, m[0]], [0, m[0], m[1], m[0] ^ m[1]]) for m in MASKS
)


def kernel(x, w_mat, scale_x, scale_w):
    m_per, k = x.shape
    _, n_per = w_mat.shape
    assert (m_per, k) == (M_PER, K)

    dot_dims = (((1,), (0,)), ((), ()))

    def body(x_hbm, w_hbm, sx_ref, sw_ref, out_ref, comm_ref, xf_ref,
             wf_ref, w8_ref, local_sems, send_sems, recv_sems, recv3_sems):
        my = lax.axis_index("i")

        cp_x = pltpu.make_async_copy(x_hbm, xf_ref, local_sems.at[0])
        cp_x.start()

        barrier_sem = pltpu.get_barrier_semaphore()
        for mask in (1, 3, 4):
            pl.semaphore_signal(
                barrier_sem, inc=1,
                device_id=(my ^ mask,), device_id_type=pl.DeviceIdType.MESH,
            )
        pl.semaphore_wait(barrier_sem, 3)

        scale = sx_ref[0] * sw_ref[0]

        def stripe_ref(off, s):
            return comm_ref.at[off, pl.ds(S_OFF[s], S_ROWS[s]), :]

        def make_rdma(s, r, off, j=None):
            m = MASKS[s][r]
            recv_sem = (
                recv3_sems.at[s, j] if r == 2 else recv_sems.at[s, r]
            )
            return pltpu.make_async_remote_copy(
                src_ref=stripe_ref(off, s),
                dst_ref=stripe_ref(off ^ m, s),
                send_sem=send_sems.at[s, r],
                recv_sem=recv_sem,
                device_id=(my ^ m,),
                device_id_type=pl.DeviceIdType.MESH,
            )

        def stripe_gemm(s, off):
            acc = lax.dot_general(
                stripe_ref(off, s)[...], w8_ref[...], dot_dims,
                preferred_element_type=jnp.float32,
            )
            row0 = (my ^ off) * M_PER + S_OFF[s]
            out_ref[pl.ds(row0, S_ROWS[s]), :] = jnp.maximum(
                acc * scale, 0.0
            )

        cp_x.wait()
        comm_ref[0] = xf_ref[...].astype(jnp.float8_e4m3fn)
        r0 = []
        for s in range(3):
            rdma = make_rdma(s, 0, 0)
            rdma.start()
            r0.append(rdma)

        for half in range(2):
            cp_w = pltpu.make_async_copy(
                w_hbm.at[pl.ds(half * (K // 2), K // 2), :],
                wf_ref, local_sems.at[1 + half],
            )
            cp_w.start()
            cp_w.wait()
            w8_ref[pl.ds(half * (K // 2), K // 2), :] = wf_ref[...].astype(
                jnp.float8_e5m2
            )

        for rdma in r0:
            rdma.wait()

        r1 = []
        for s in range(3):
            for off in OFFS[s][1]:
                rdma = make_rdma(s, 1, off)
                rdma.start()
                r1.append(rdma)
        acc = lax.dot_general(
            comm_ref[0], w8_ref[...], dot_dims,
            preferred_element_type=jnp.float32,
        )
        out_ref[pl.ds(my * M_PER, M_PER), :] = jnp.maximum(acc * scale, 0.0)
        for s in range(3):
            stripe_gemm(s, MASKS[s][0])
        for rdma in r1:
            rdma.wait()

        r2 = [[None] * 4 for _ in range(3)]
        for s in range(3):
            for j, off in enumerate(OFFS[s][2]):
                rdma = make_rdma(s, 2, off, j=j)
                rdma.start()
                r2[s][j] = rdma
        for s in range(3):
            m0, m1 = MASKS[s][0], MASKS[s][1]
            stripe_gemm(s, m1)
            stripe_gemm(s, m0 ^ m1)
        for j in range(4):
            for s in (1, 2, 0):
                r2[s][j].wait()
                stripe_gemm(s, MASKS[s][2] ^ OFFS[s][2][j])

    return pl.pallas_call(
        body,
        out_shape=jax.ShapeDtypeStruct((N_DEV * m_per, n_per), jnp.float32),
        in_specs=[
            pl.BlockSpec(memory_space=pl.ANY),
            pl.BlockSpec(memory_space=pl.ANY),
            pl.BlockSpec(memory_space=pltpu.SMEM),
            pl.BlockSpec(memory_space=pltpu.SMEM),
        ],
        out_specs=pl.BlockSpec(memory_space=pltpu.VMEM),
        scratch_shapes=[
            pltpu.VMEM((N_DEV, M_PER, K), jnp.float8_e4m3fn),
            pltpu.VMEM((M_PER, K), jnp.float32),
            pltpu.VMEM((K // 2, n_per), jnp.float32),
            pltpu.VMEM((K, n_per), jnp.float8_e5m2),
            pltpu.SemaphoreType.DMA((3,)),
            pltpu.SemaphoreType.DMA((3, 3)),
            pltpu.SemaphoreType.DMA((3, 2)),
            pltpu.SemaphoreType.DMA((3, 4)),
        ],
        compiler_params=pltpu.CompilerParams(
            collective_id=0,
            vmem_limit_bytes=100 * 1024 * 1024,
        ),
    )(x, w_mat, scale_x, scale_w)


# device time: 84658 ns/iter; 2.7266x vs baseline; 1.0768x over previous
import jax
import jax.numpy as jnp
from jax import lax
from jax.experimental import pallas as pl
from jax.experimental.pallas import tpu as pltpu

N_DEV = 8
M_PER = 512
K = 4096

S_OFF = (0, 192, 352)
S_ROWS = (192, 160, 160)
MASKS = ((1, 3, 4), (3, 4, 1), (4, 1, 3))
OFFS = tuple(
    ([0], [0, m[0]], [0, m[0], m[1], m[0] ^ m[1]]) for m in MASKS
)


def kernel(x, w_mat, scale_x, scale_w):
    m_per, k = x.shape
    _, n_per = w_mat.shape
    assert (m_per, k) == (M_PER, K)

    dot_dims = (((1,), (0,)), ((), ()))

    def body(x_hbm, w_hbm, sx_ref, sw_ref, out_hbm, comm_ref, xf_ref,
             wf_ref, w8_ref, ob_ref, local_sems, out_sems,
             send_sems, recv_sems, recv3_sems):
        my = lax.axis_index("i")

        cp_x = []
        for s in range(3):
            cp = pltpu.make_async_copy(
                x_hbm.at[pl.ds(S_OFF[s], S_ROWS[s]), :],
                xf_ref.at[pl.ds(S_OFF[s], S_ROWS[s]), :],
                local_sems.at[s],
            )
            cp.start()
            cp_x.append(cp)

        barrier_sem = pltpu.get_barrier_semaphore()
        for mask in (1, 3, 4):
            pl.semaphore_signal(
                barrier_sem, inc=1,
                device_id=(my ^ mask,), device_id_type=pl.DeviceIdType.MESH,
            )
        pl.semaphore_wait(barrier_sem, 3)

        scale = sx_ref[0] * sw_ref[0]

        def stripe_ref(off, s):
            return comm_ref.at[off, pl.ds(S_OFF[s], S_ROWS[s]), :]

        def make_rdma(s, r, off, j=None):
            m = MASKS[s][r]
            recv_sem = (
                recv3_sems.at[s, j] if r == 2 else recv_sems.at[s, r]
            )
            return pltpu.make_async_remote_copy(
                src_ref=stripe_ref(off, s),
                dst_ref=stripe_ref(off ^ m, s),
                send_sem=send_sems.at[s, r],
                recv_sem=recv_sem,
                device_id=(my ^ m,),
                device_id_type=pl.DeviceIdType.MESH,
            )

        ob_last = [0, 0]
        ob_slot = [0]

        def ob_wait(slot):
            if ob_last[slot]:
                pltpu.make_async_copy(
                    ob_ref.at[slot, pl.ds(0, ob_last[slot]), :],
                    out_hbm.at[pl.ds(0, ob_last[slot]), :],
                    out_sems.at[slot],
                ).wait()
                ob_last[slot] = 0

        def emit_rows(acc, row0, rows):
            slot = ob_slot[0]
            ob_slot[0] ^= 1
            ob_wait(slot)
            ob_ref[slot, pl.ds(0, rows), :] = jnp.maximum(acc * scale, 0.0)
            pltpu.make_async_copy(
                ob_ref.at[slot, pl.ds(0, rows), :],
                out_hbm.at[pl.ds(row0, rows), :],
                out_sems.at[slot],
            ).start()
            ob_last[slot] = rows

        def stripe_gemm(s, off):
            acc = lax.dot_general(
                stripe_ref(off, s)[...], w8_ref[...], dot_dims,
                preferred_element_type=jnp.float32,
            )
            emit_rows(acc, (my ^ off) * M_PER + S_OFF[s], S_ROWS[s])

        r0 = []
        for s in range(3):
            cp_x[s].wait()
            comm_ref[0, pl.ds(S_OFF[s], S_ROWS[s]), :] = xf_ref[
                pl.ds(S_OFF[s], S_ROWS[s]), :
            ].astype(jnp.float8_e4m3fn)
            rdma = make_rdma(s, 0, 0)
            rdma.start()
            r0.append(rdma)

        for half in range(2):
            cp_w = pltpu.make_async_copy(
                w_hbm.at[pl.ds(half * (K // 2), K // 2), :],
                wf_ref, local_sems.at[3 + half],
            )
            cp_w.start()
            cp_w.wait()
            w8_ref[pl.ds(half * (K // 2), K // 2), :] = wf_ref[...].astype(
                jnp.float8_e5m2
            )

        for rdma in r0:
            rdma.wait()

        r1 = []
        for s in range(3):
            for off in OFFS[s][1]:
                rdma = make_rdma(s, 1, off)
                rdma.start()
                r1.append(rdma)
        acc = lax.dot_general(
            comm_ref[0], w8_ref[...], dot_dims,
            preferred_element_type=jnp.float32,
        )
        emit_rows(acc, my * M_PER, M_PER)
        for s in range(3):
            stripe_gemm(s, MASKS[s][0])
        for rdma in r1:
            rdma.wait()

        r2 = [[None] * 4 for _ in range(3)]
        for s in range(3):
            for j, off in enumerate(OFFS[s][2]):
                rdma = make_rdma(s, 2, off, j=j)
                rdma.start()
                r2[s][j] = rdma
        for s in range(3):
            m0, m1 = MASKS[s][0], MASKS[s][1]
            stripe_gemm(s, m1)
            stripe_gemm(s, m0 ^ m1)
        for j in range(4):
            for s in (1, 2, 0):
                r2[s][j].wait()
                stripe_gemm(s, MASKS[s][2] ^ OFFS[s][2][j])

        for slot in range(2):
            ob_wait(slot)

    return pl.pallas_call(
        body,
        out_shape=jax.ShapeDtypeStruct((N_DEV * m_per, n_per), jnp.float32),
        in_specs=[
            pl.BlockSpec(memory_space=pl.ANY),
            pl.BlockSpec(memory_space=pl.ANY),
            pl.BlockSpec(memory_space=pltpu.SMEM),
            pl.BlockSpec(memory_space=pltpu.SMEM),
        ],
        out_specs=pl.BlockSpec(memory_space=pl.ANY),
        scratch_shapes=[
            pltpu.VMEM((N_DEV, M_PER, K), jnp.float8_e4m3fn),
            pltpu.VMEM((M_PER, K), jnp.float32),
            pltpu.VMEM((K // 2, n_per), jnp.float32),
            pltpu.VMEM((K, n_per), jnp.float8_e5m2),
            pltpu.VMEM((2, M_PER, n_per), jnp.float32),
            pltpu.SemaphoreType.DMA((5,)),
            pltpu.SemaphoreType.DMA((2,)),
            pltpu.SemaphoreType.DMA((3, 3)),
            pltpu.SemaphoreType.DMA((3, 2)),
            pltpu.SemaphoreType.DMA((3, 4)),
        ],
        compiler_params=pltpu.CompilerParams(
            collective_id=0,
            vmem_limit_bytes=100 * 1024 * 1024,
        ),
    )(x, w_mat, scale_x, scale_w)
